# baseline (speedup 1.0000x reference)
import functools

import jax
import jax.numpy as jnp
from jax import lax
from jax.experimental import pallas as pl
from jax.experimental.pallas import tpu as pltpu
from jax.experimental.pallas import tpu_sc as plsc

_NBUF = 4


def _gather_kernel(n_total, d_model, b_per_w, chunk, n_chunks):
    mesh = plsc.VectorSubcoreMesh(core_axis_name="c", subcore_axis_name="s")

    @functools.partial(
        pl.kernel,
        mesh=mesh,
        out_type=jax.ShapeDtypeStruct((n_total, d_model), jnp.float32),
        scratch_types=[
            pltpu.VMEM((n_chunks, chunk), jnp.int32),
            pltpu.VMEM((_NBUF, chunk, d_model), jnp.float32),
            pltpu.SemaphoreType.DMA((_NBUF,)),
            pltpu.SemaphoreType.DMA((_NBUF,)),
        ],
    )
    def k(table_hbm, idx_hbm, out_hbm, idx_v, rows_v, gsem, ssem):
        nc = plsc.get_sparse_core_info().num_cores
        wid = lax.axis_index("s") * nc + lax.axis_index("c")
        base = wid * b_per_w
        pltpu.sync_copy(idx_hbm.at[wid], idx_v)

        def gather(c, b):
            cp = pltpu.make_async_copy(
                table_hbm.at[idx_v.at[c]], rows_v.at[b], gsem.at[b]
            )
            cp.start()
            return cp

        def scatter(c, b):
            pltpu.make_async_copy(
                table_hbm.at[idx_v.at[c]], rows_v.at[b], gsem.at[b]
            ).wait()
            cp = pltpu.make_async_copy(
                rows_v.at[b],
                out_hbm.at[pl.ds(base + c * chunk, chunk)],
                ssem.at[b],
            )
            cp.start()
            return cp

        def wait_scatter(c, b):
            pltpu.make_async_copy(
                rows_v.at[b],
                out_hbm.at[pl.ds(base + c * chunk, chunk)],
                ssem.at[b],
            ).wait()

        for c in range(_NBUF):
            gather(c, c)
        scatter(0, 0)

        n_steady = n_chunks - _NBUF
        assert n_steady % _NBUF == 0

        def body(j):
            c0 = 1 + j * _NBUF
            for u in range(_NBUF):
                c = c0 + u
                scatter(c, (1 + u) % _NBUF)
                wait_scatter(c - 1, u % _NBUF)
                gather(c + 3, u % _NBUF)

        pl.loop(0, n_steady // _NBUF)(body)

        for c in range(n_chunks - 3, n_chunks):
            scatter(c, c % _NBUF)
            wait_scatter(c - 1, (c - 1) % _NBUF)
        wait_scatter(n_chunks - 1, (n_chunks - 1) % _NBUF)

    return k


def _tc_gather(n_rows, d_model, group):
    n_steps = n_rows // group

    def body(idx_ref, *refs):
        row_refs = refs[:group]
        out_ref = refs[group]
        for j in range(group):
            out_ref[j, :] = row_refs[j][0, 0, :]

    grid_spec = pltpu.PrefetchScalarGridSpec(
        num_scalar_prefetch=1,
        grid=(n_steps,),
        in_specs=[
            pl.BlockSpec((1, 1, d_model), lambda i, idx_ref, j=j: (idx_ref[i * group + j], 0, 0))
            for j in range(group)
        ],
        out_specs=pl.BlockSpec((group, d_model), lambda i, idx_ref: (i, 0)),
    )
    return pl.pallas_call(
        body,
        grid_spec=grid_spec,
        out_shape=jax.ShapeDtypeStruct((n_rows, d_model), jnp.float32),
    )


def kernel(x, pe):
    b, s = x.shape
    v, d = pe.shape
    info = plsc.get_sparse_core_info()
    nw = info.num_cores * info.num_subcores

    b_sc = 3
    n_sc = b_sc * s
    n_tc = (b - b_sc) * s

    b_per_w = n_sc // nw
    chunk = 16
    n_chunks = b_per_w // chunk
    xi = x.astype(jnp.int32)
    idx3 = xi[:b_sc].reshape(nw, n_chunks, chunk)
    k = _gather_kernel(n_sc, d, b_per_w, chunk, n_chunks)
    out_sc = k(pe, idx3)

    tc = _tc_gather(n_tc, d, 8)
    pe3 = pe.reshape(v, 1, d)
    out_tc = tc(xi[b_sc:].reshape(n_tc), *([pe3] * 8))

    out = jnp.concatenate([out_sc.reshape(b_sc, s, d), out_tc.reshape(b - b_sc, s, d)], axis=0)
    return out
